# Initial kernel scaffold; baseline (speedup 1.0000x reference)
import jax
import jax.numpy as jnp
from jax.experimental import pallas as pl


def kernel(x_categorical, x_numerical, emb_tables, bn_num_g, bn_num_b, W1, b1, g1, be1, W2, b2, g2, be2, W3, b3):
    raise NotImplementedError("write your pallas kernel here")


# speedup vs baseline: 2.1926x; 2.1926x over previous
import functools

import jax
import jax.numpy as jnp
from jax import lax
from jax.experimental import pallas as pl
from jax.experimental.pallas import tpu as pltpu
from jax.experimental.pallas import tpu_sc as plsc

B = 4096
F = 26
V = 100000
D = 32
NUM = 13
H1 = 512
H2 = 256
OUT = 100
EPS = 1e-5
NUMP = 128

_NC, _NS = 2, 16
_NW = _NC * _NS
_BT = B * F
_BPW = _BT // _NW

@functools.cache
def _make_sc_gather():
    mesh = plsc.VectorSubcoreMesh(
        core_axis_name="c", subcore_axis_name="s")

    @functools.partial(
        pl.kernel,
        mesh=mesh,
        out_type=jax.ShapeDtypeStruct((_BT, D), jnp.float32),
        compiler_params=pltpu.CompilerParams(use_tc_tiling_on_sc=False),
        scratch_types=[
            pltpu.VMEM((_BPW,), jnp.int32),
            pltpu.VMEM((_BPW, D), jnp.float32),
            pltpu.SemaphoreType.DMA,
        ],
    )
    def _sc_gather(table_hbm, idx_hbm, out_hbm, idx_v, rows_v, sem):
        wid = lax.axis_index("s") * _NC + lax.axis_index("c")
        base = wid * _BPW
        pltpu.sync_copy(idx_hbm.at[pl.ds(base, _BPW)], idx_v)
        pltpu.async_copy(table_hbm.at[idx_v], rows_v, sem).wait()
        pltpu.sync_copy(rows_v, out_hbm.at[pl.ds(base, _BPW)])

    return _sc_gather


def _mlp_body(emb_ref, xn_ref, gn_ref, bn_ref, w1a_ref, w1b_ref, b1_ref,
              g1_ref, be1_ref, w2_ref, b2_ref, g2_ref, be2_ref,
              w3_ref, b3_ref, out_ref):
    xn = xn_ref[...]
    m = jnp.mean(xn, axis=0, keepdims=True)
    v = jnp.mean((xn - m) * (xn - m), axis=0, keepdims=True)
    xn = gn_ref[...] * (xn - m) * lax.rsqrt(v + EPS) + bn_ref[...]

    h = jnp.dot(emb_ref[...], w1a_ref[...], preferred_element_type=jnp.float32)
    h = h + jnp.dot(xn, w1b_ref[...], preferred_element_type=jnp.float32)
    h = jnp.maximum(h + b1_ref[...], 0.0)
    m1 = jnp.mean(h, axis=0, keepdims=True)
    v1 = jnp.mean((h - m1) * (h - m1), axis=0, keepdims=True)
    h = g1_ref[...] * (h - m1) * lax.rsqrt(v1 + EPS) + be1_ref[...]

    h2 = jnp.dot(h, w2_ref[...], preferred_element_type=jnp.float32)
    h2 = jnp.maximum(h2 + b2_ref[...], 0.0)
    m2 = jnp.mean(h2, axis=0, keepdims=True)
    v2 = jnp.mean((h2 - m2) * (h2 - m2), axis=0, keepdims=True)
    h2 = g2_ref[...] * (h2 - m2) * lax.rsqrt(v2 + EPS) + be2_ref[...]

    out_ref[...] = (
        jnp.dot(h2, w3_ref[...], preferred_element_type=jnp.float32)
        + b3_ref[...]
    )


def kernel(x_categorical, x_numerical, emb_tables, bn_num_g, bn_num_b,
           W1, b1, g1, be1, W2, b2, g2, be2, W3, b3):
    offs = (jnp.arange(F, dtype=jnp.int32) * V)[None, :]
    idx = (x_categorical.astype(jnp.int32) + offs).reshape(_BT)
    table = emb_tables.reshape(F * V, D)
    emb = _make_sc_gather()(table, idx).reshape(B, F * D)

    xn = jnp.pad(x_numerical, ((0, 0), (0, NUMP - NUM)))
    gn = jnp.pad(bn_num_g, (0, NUMP - NUM)).reshape(1, NUMP)
    bn = jnp.pad(bn_num_b, (0, NUMP - NUM)).reshape(1, NUMP)
    w1a = W1[:, :F * D].T
    w1b = jnp.pad(W1[:, F * D:], ((0, 0), (0, NUMP - NUM))).T

    return pl.pallas_call(
        _mlp_body,
        out_shape=jax.ShapeDtypeStruct((B, OUT), jnp.float32),
    )(emb, xn, gn, bn, w1a, w1b, b1.reshape(1, H1),
      g1.reshape(1, H1), be1.reshape(1, H1), W2.T, b2.reshape(1, H2),
      g2.reshape(1, H2), be2.reshape(1, H2), W3.T, b3.reshape(1, OUT))
